# Initial kernel scaffold; baseline (speedup 1.0000x reference)
import jax
import jax.numpy as jnp
from jax.experimental import pallas as pl


def kernel(x, p, w1, b1, w2, b2, w3, b3, wc, bc):
    raise NotImplementedError("write your pallas kernel here")


# speedup vs baseline: 2.0620x; 2.0620x over previous
import functools

import jax
import jax.numpy as jnp
from jax import lax
from jax.experimental import pallas as pl
from jax.experimental.pallas import tpu as pltpu


def _conv_relu(flat, wt, b, Wp, Rout, row0=None, H=None):
    Co = wt.shape[2]
    Lout = Rout * Wp
    fpad = jnp.pad(flat, ((1, 1), (0, 0)))
    acc = None
    for t in range(9):
        ky, kx = t // 3, t % 3
        start = 1 + Wp + (ky - 1) * Wp + (kx - 1)
        a = fpad[start:start + Lout]
        d = lax.dot_general(a, wt[t], (((1,), (0,)), ((), ())),
                            preferred_element_type=jnp.float32)
        acc = d if acc is None else acc + d
    h = jnp.maximum(acc + b, 0.0)
    idx = lax.broadcasted_iota(jnp.int32, (Lout, 1), 0)
    col = idx % Wp
    mask = (col >= 1) & (col <= Wp - 2)
    if row0 is not None:
        row = idx // Wp + row0
        mask = mask & (row >= 3) & (row <= H + 2)
    return h * mask.astype(jnp.float32)


def _trunk_kernel(xp_ref, w1_ref, w2_ref, w3_ref, b1_ref, b2_ref, b3_ref,
                  val_ref, psum_ref, *, T, W, H):
    Wp = W + 2
    t = pl.program_id(1)
    a = t * T
    xc = xp_ref[0, :, pl.ds(a, T + 6), :]
    xf = jnp.transpose(xc.reshape(3, (T + 6) * Wp))
    h1 = _conv_relu(xf, w1_ref[...], b1_ref[...], Wp, T + 4,
                    row0=a + 1, H=H)
    h2 = _conv_relu(h1, w2_ref[...], b2_ref[...], Wp, T + 2,
                    row0=a + 2, H=H)
    o = _conv_relu(h2, w3_ref[...], b3_ref[...], Wp, T)
    v = o.sum(axis=1).reshape(T, Wp)
    val_ref[...] = v[:, 1:W + 1].reshape(1, T, W)
    s = o.sum(axis=0).reshape(1, 1, -1)

    @pl.when(t == 0)
    def _():
        psum_ref[...] = s

    @pl.when(t != 0)
    def _():
        psum_ref[...] += s


def _head_kernel(p_ref, psum_ref, wct_ref, bc_ref, out1_ref, counts_ref, *,
                 npix):
    C3 = counts_ref.shape[1]
    ncols = p_ref.shape[1]
    bins = lax.broadcasted_iota(jnp.int32, (1, C3), 1)
    total = jnp.zeros((1, C3), jnp.float32)
    for j in range(ncols):
        col = p_ref[:, j:j + 1]
        total = total + jnp.sum((col == bins).astype(jnp.float32),
                                axis=0, keepdims=True)
    counts_ref[...] = total + 1.0
    pooled = psum_ref[:, 0, :] * (1.0 / npix)
    logits = lax.dot_general(pooled, wct_ref[...], (((1,), (0,)), ((), ())),
                             preferred_element_type=jnp.float32)
    logits = logits + bc_ref[...]
    m = jnp.max(logits, axis=1, keepdims=True)
    e = jnp.exp(logits - m)
    out1_ref[...] = e / jnp.sum(e, axis=1, keepdims=True)


def _writer_kernel(val_ref, counts_ref, out_ref):
    v = val_ref[...]
    inv = 1.0 / counts_ref[...]
    out_ref[...] = v[:, None, :, :] * inv


@jax.jit
def kernel(x, p, w1, b1, w2, b2, w3, b3, wc, bc):
    B, _, H, W = x.shape
    C1 = w1.shape[0]
    C2 = w2.shape[0]
    C3 = w3.shape[0]
    Wp = W + 2
    T = next(t for t in (32, 56, 16, 8, H) if H % t == 0 and
             (t % 8 == 0 or t == H))
    NT = H // T

    xp = jnp.pad(x, ((0, 0), (0, 0), (3, 3), (1, 1)))
    w1t = jnp.transpose(w1, (2, 3, 1, 0)).reshape(9, 3, C1)
    w2t = jnp.transpose(w2, (2, 3, 1, 0)).reshape(9, C1, C2)
    w3t = jnp.transpose(w3, (2, 3, 1, 0)).reshape(9, C2, C3)
    b1r = b1.reshape(1, C1)
    b2r = b2.reshape(1, C2)
    b3r = b3.reshape(1, C3)
    wct = jnp.transpose(wc)
    bcr = bc.reshape(1, 2)

    val, psum = pl.pallas_call(
        functools.partial(_trunk_kernel, T=T, W=W, H=H),
        grid=(B, NT),
        in_specs=[
            pl.BlockSpec((1, 3, H + 6, Wp), lambda b, t: (b, 0, 0, 0)),
            pl.BlockSpec((9, 3, C1), lambda b, t: (0, 0, 0)),
            pl.BlockSpec((9, C1, C2), lambda b, t: (0, 0, 0)),
            pl.BlockSpec((9, C2, C3), lambda b, t: (0, 0, 0)),
            pl.BlockSpec((1, C1), lambda b, t: (0, 0)),
            pl.BlockSpec((1, C2), lambda b, t: (0, 0)),
            pl.BlockSpec((1, C3), lambda b, t: (0, 0)),
        ],
        out_specs=[
            pl.BlockSpec((1, T, W), lambda b, t: (b, t, 0)),
            pl.BlockSpec((1, 1, C3), lambda b, t: (b, 0, 0)),
        ],
        out_shape=[
            jax.ShapeDtypeStruct((B, H, W), jnp.float32),
            jax.ShapeDtypeStruct((B, 1, C3), jnp.float32),
        ],
        compiler_params=pltpu.CompilerParams(
            dimension_semantics=("parallel", "arbitrary"),
            vmem_limit_bytes=48 * 1024 * 1024,
        ),
    )(xp, w1t, w2t, w3t, b1r, b2r, b3r)

    p8 = p.reshape(H * W // 8, 8)
    out1, counts = pl.pallas_call(
        functools.partial(_head_kernel, npix=float(H * W)),
        out_shape=[
            jax.ShapeDtypeStruct((B, 2), jnp.float32),
            jax.ShapeDtypeStruct((1, C3), jnp.float32),
        ],
    )(p8, psum, wct, bcr)
    counts = counts.reshape(1, C3, 1, 1)

    result = pl.pallas_call(
        _writer_kernel,
        grid=(B, NT),
        in_specs=[
            pl.BlockSpec((1, T, W), lambda b, t: (b, t, 0)),
            pl.BlockSpec((1, C3, 1, 1), lambda b, t: (0, 0, 0, 0)),
        ],
        out_specs=pl.BlockSpec((1, C3, T, W), lambda b, t: (b, 0, t, 0)),
        out_shape=jax.ShapeDtypeStruct((B, C3, H, W), jnp.float32),
        compiler_params=pltpu.CompilerParams(
            dimension_semantics=("parallel", "arbitrary"),
        ),
    )(val, counts)

    return out1, result


# speedup vs baseline: 2.0805x; 1.0090x over previous
import functools

import jax
import jax.numpy as jnp
from jax import lax
from jax.experimental import pallas as pl
from jax.experimental.pallas import tpu as pltpu


def _conv_relu(flat, wt, b, Wp, Rout, row0=None, H=None):
    Co = wt.shape[2]
    Lout = Rout * Wp
    fpad = jnp.pad(flat, ((1, 1), (0, 0))).astype(jnp.bfloat16)
    acc = None
    for t in range(9):
        ky, kx = t // 3, t % 3
        start = 1 + Wp + (ky - 1) * Wp + (kx - 1)
        a = fpad[start:start + Lout]
        d = lax.dot_general(a, wt[t], (((1,), (0,)), ((), ())),
                            preferred_element_type=jnp.float32)
        acc = d if acc is None else acc + d
    h = jnp.maximum(acc + b, 0.0)
    idx = lax.broadcasted_iota(jnp.int32, (Lout, 1), 0)
    col = idx % Wp
    mask = (col >= 1) & (col <= Wp - 2)
    if row0 is not None:
        row = idx // Wp + row0
        mask = mask & (row >= 3) & (row <= H + 2)
    return h * mask.astype(jnp.float32)


def _trunk_kernel(xp_ref, w1_ref, w2_ref, w3_ref, b1_ref, b2_ref, b3_ref,
                  val_ref, psum_ref, *, T, W, H):
    Wp = W + 2
    t = pl.program_id(1)
    a = t * T
    xc = xp_ref[0, :, pl.ds(a, T + 6), :]
    xf = jnp.transpose(xc.reshape(3, (T + 6) * Wp))
    h1 = _conv_relu(xf, w1_ref[...], b1_ref[...], Wp, T + 4,
                    row0=a + 1, H=H)
    h2 = _conv_relu(h1, w2_ref[...], b2_ref[...], Wp, T + 2,
                    row0=a + 2, H=H)
    o = _conv_relu(h2, w3_ref[...], b3_ref[...], Wp, T)
    v = o.sum(axis=1).reshape(T, Wp)
    val_ref[...] = v[:, 1:W + 1].reshape(1, T, W)
    s = o.sum(axis=0).reshape(1, 1, -1)

    @pl.when(t == 0)
    def _():
        psum_ref[...] = s

    @pl.when(t != 0)
    def _():
        psum_ref[...] += s


def _head_kernel(p_ref, psum_ref, wct_ref, bc_ref, out1_ref, counts_ref, *,
                 npix):
    C3 = counts_ref.shape[1]
    ncols = p_ref.shape[1]
    bins = lax.broadcasted_iota(jnp.int32, (1, C3), 1)
    total = jnp.zeros((1, C3), jnp.float32)
    for j in range(ncols):
        col = p_ref[:, j:j + 1]
        total = total + jnp.sum((col == bins).astype(jnp.float32),
                                axis=0, keepdims=True)
    counts_ref[...] = total + 1.0
    pooled = psum_ref[:, 0, :] * (1.0 / npix)
    logits = lax.dot_general(pooled, wct_ref[...], (((1,), (0,)), ((), ())),
                             preferred_element_type=jnp.float32)
    logits = logits + bc_ref[...]
    m = jnp.max(logits, axis=1, keepdims=True)
    e = jnp.exp(logits - m)
    out1_ref[...] = e / jnp.sum(e, axis=1, keepdims=True)


def _writer_kernel(val_ref, counts_ref, out_ref):
    v = val_ref[...]
    inv = 1.0 / counts_ref[...]
    out_ref[...] = v[:, None, :, :] * inv


@jax.jit
def kernel(x, p, w1, b1, w2, b2, w3, b3, wc, bc):
    B, _, H, W = x.shape
    C1 = w1.shape[0]
    C2 = w2.shape[0]
    C3 = w3.shape[0]
    Wp = W + 2
    T = next(t for t in (32, 56, 16, 8, H) if H % t == 0 and
             (t % 8 == 0 or t == H))
    NT = H // T

    xp = jnp.pad(x, ((0, 0), (0, 0), (3, 3), (1, 1)))
    w1t = jnp.transpose(w1, (2, 3, 1, 0)).reshape(9, 3, C1).astype(jnp.bfloat16)
    w2t = jnp.transpose(w2, (2, 3, 1, 0)).reshape(9, C1, C2).astype(jnp.bfloat16)
    w3t = jnp.transpose(w3, (2, 3, 1, 0)).reshape(9, C2, C3).astype(jnp.bfloat16)
    b1r = b1.reshape(1, C1)
    b2r = b2.reshape(1, C2)
    b3r = b3.reshape(1, C3)
    wct = jnp.transpose(wc)
    bcr = bc.reshape(1, 2)

    val, psum = pl.pallas_call(
        functools.partial(_trunk_kernel, T=T, W=W, H=H),
        grid=(B, NT),
        in_specs=[
            pl.BlockSpec((1, 3, H + 6, Wp), lambda b, t: (b, 0, 0, 0)),
            pl.BlockSpec((9, 3, C1), lambda b, t: (0, 0, 0)),
            pl.BlockSpec((9, C1, C2), lambda b, t: (0, 0, 0)),
            pl.BlockSpec((9, C2, C3), lambda b, t: (0, 0, 0)),
            pl.BlockSpec((1, C1), lambda b, t: (0, 0)),
            pl.BlockSpec((1, C2), lambda b, t: (0, 0)),
            pl.BlockSpec((1, C3), lambda b, t: (0, 0)),
        ],
        out_specs=[
            pl.BlockSpec((1, T, W), lambda b, t: (b, t, 0)),
            pl.BlockSpec((1, 1, C3), lambda b, t: (b, 0, 0)),
        ],
        out_shape=[
            jax.ShapeDtypeStruct((B, H, W), jnp.float32),
            jax.ShapeDtypeStruct((B, 1, C3), jnp.float32),
        ],
        compiler_params=pltpu.CompilerParams(
            dimension_semantics=("parallel", "arbitrary"),
            vmem_limit_bytes=48 * 1024 * 1024,
        ),
    )(xp, w1t, w2t, w3t, b1r, b2r, b3r)

    p8 = p.reshape(H * W // 8, 8)
    out1, counts = pl.pallas_call(
        functools.partial(_head_kernel, npix=float(H * W)),
        out_shape=[
            jax.ShapeDtypeStruct((B, 2), jnp.float32),
            jax.ShapeDtypeStruct((1, C3), jnp.float32),
        ],
    )(p8, psum, wct, bcr)
    counts = counts.reshape(1, C3, 1, 1)

    result = pl.pallas_call(
        _writer_kernel,
        grid=(B, NT),
        in_specs=[
            pl.BlockSpec((1, T, W), lambda b, t: (b, t, 0)),
            pl.BlockSpec((1, C3, 1, 1), lambda b, t: (0, 0, 0, 0)),
        ],
        out_specs=pl.BlockSpec((1, C3, T, W), lambda b, t: (b, 0, t, 0)),
        out_shape=jax.ShapeDtypeStruct((B, C3, H, W), jnp.float32),
        compiler_params=pltpu.CompilerParams(
            dimension_semantics=("parallel", "arbitrary"),
        ),
    )(val, counts)

    return out1, result
